# baseline (speedup 1.0000x reference)
import jax
import jax.numpy as jnp
from jax.experimental import pallas as pl
from jax.experimental.pallas import tpu as pltpu

_B_BLOCK = 1024
_K = 1024
_DIM = 256
_NCB = 4


def _rvq_body(q_ref, hi_ref, cbs_ref, idx_ref, emb_ref):
    q = q_ref[...]
    r = q
    col = jax.lax.broadcasted_iota(jnp.int32, (q.shape[0], _K), 1)
    col3 = jax.lax.broadcasted_iota(jnp.int32, (q.shape[0], 3 * _K), 1)
    col3 = jax.lax.bitwise_and(col3, _K - 1)
    for i in range(_NCB):
        sims = jax.lax.dot_general(
            r.astype(jnp.bfloat16), hi_ref[i],
            (((1,), (1,)), ((), ())),
            preferred_element_type=jnp.float32)
        m = jnp.max(sims, axis=1, keepdims=True)
        idx = jnp.min(jnp.where(sims == m, col, _K), axis=1).astype(jnp.int32)
        onehot = (col == idx[:, None]).astype(jnp.float32)
        sel = jax.lax.dot_general(
            onehot, cbs_ref[i], (((1,), (0,)), ((), ())),
            precision=jax.lax.Precision.HIGHEST,
            preferred_element_type=jnp.float32)
        r = r - sel
        idx_ref[i, :] = idx
    emb_ref[...] = q - r


@jax.jit
def kernel(query, cb0, cb1, cb2, cb3):
    B = query.shape[0]
    cbs = jnp.stack([cb0, cb1, cb2, cb3], axis=0)
    hi = cbs.astype(jnp.bfloat16)
    grid = (B // _B_BLOCK,)
    idx, emb = pl.pallas_call(
        _rvq_body,
        grid=grid,
        in_specs=[
            pl.BlockSpec((_B_BLOCK, _DIM), lambda i: (i, 0)),
            pl.BlockSpec((_NCB, _K, _DIM), lambda i: (0, 0, 0)),
            pl.BlockSpec((_NCB, _K, _DIM), lambda i: (0, 0, 0)),
        ],
        out_specs=[
            pl.BlockSpec((_NCB, _B_BLOCK), lambda i: (0, i)),
            pl.BlockSpec((_B_BLOCK, _DIM), lambda i: (i, 0)),
        ],
        out_shape=[
            jax.ShapeDtypeStruct((_NCB, B), jnp.int32),
            jax.ShapeDtypeStruct((B, _DIM), jnp.float32),
        ],
        compiler_params=pltpu.CompilerParams(
            dimension_semantics=("arbitrary",),
        ),
    )(query, hi, cbs)
    return idx, emb


# speedup vs baseline: 1.6066x; 1.6066x over previous
import jax
import jax.numpy as jnp
from jax.experimental import pallas as pl
from jax.experimental.pallas import tpu as pltpu

_B_BLOCK = 1024
_K = 1024
_DIM = 256
_NCB = 4


def _rvq_body(q_ref, hi_ref, cbcat_ref, idx_ref, emb_ref):
    q = q_ref[...]
    r = q
    col = jax.lax.broadcasted_iota(jnp.int32, (q.shape[0], _K), 1)
    col3 = jax.lax.broadcasted_iota(jnp.int32, (q.shape[0], 3 * _K), 1)
    col3 = jax.lax.bitwise_and(col3, _K - 1)
    for i in range(_NCB):
        sims = jax.lax.dot_general(
            r.astype(jnp.bfloat16), hi_ref[i], (((1,), (1,)), ((), ())),
            preferred_element_type=jnp.float32)
        m = jnp.max(sims, axis=1, keepdims=True)
        idx = jnp.min(jnp.where(sims == m, col, _K), axis=1).astype(jnp.int32)
        onehot3 = (col3 == idx[:, None]).astype(jnp.bfloat16)
        sel = jax.lax.dot_general(
            onehot3, cbcat_ref[i], (((1,), (0,)), ((), ())),
            preferred_element_type=jnp.float32)
        r = r - sel
        idx_ref[i, :] = idx
    emb_ref[...] = q - r


@jax.jit
def kernel(query, cb0, cb1, cb2, cb3):
    B = query.shape[0]
    cbs = jnp.stack([cb0, cb1, cb2, cb3], axis=0)
    hi = cbs.astype(jnp.bfloat16)
    m16 = jnp.int32(-65536)
    t1f = jax.lax.bitcast_convert_type(
        jax.lax.bitcast_convert_type(cbs, jnp.int32) & m16, jnp.float32)
    r1 = cbs - t1f
    t2f = jax.lax.bitcast_convert_type(
        jax.lax.bitcast_convert_type(r1, jnp.int32) & m16, jnp.float32)
    r2 = r1 - t2f
    cbcat = jnp.concatenate(
        [t1f.astype(jnp.bfloat16), t2f.astype(jnp.bfloat16),
         r2.astype(jnp.bfloat16)], axis=1)
    grid = (B // _B_BLOCK,)
    idx, emb = pl.pallas_call(
        _rvq_body,
        grid=grid,
        in_specs=[
            pl.BlockSpec((_B_BLOCK, _DIM), lambda i: (i, 0)),
            pl.BlockSpec((_NCB, _K, _DIM), lambda i: (0, 0, 0)),
            pl.BlockSpec((_NCB, 3 * _K, _DIM), lambda i: (0, 0, 0)),
        ],
        out_specs=[
            pl.BlockSpec((_NCB, _B_BLOCK), lambda i: (0, i)),
            pl.BlockSpec((_B_BLOCK, _DIM), lambda i: (i, 0)),
        ],
        out_shape=[
            jax.ShapeDtypeStruct((_NCB, B), jnp.int32),
            jax.ShapeDtypeStruct((B, _DIM), jnp.float32),
        ],
        compiler_params=pltpu.CompilerParams(
            dimension_semantics=("arbitrary",),
        ),
    )(query, hi, cbcat)
    return idx, emb


# speedup vs baseline: 2.4624x; 1.5327x over previous
import jax
import jax.numpy as jnp
from jax.experimental import pallas as pl
from jax.experimental.pallas import tpu as pltpu

_B_BLOCK = 1024
_NSPLIT = 2
_K = 1024
_DIM = 256
_NCB = 4


def _round(r, col, hi, cbcat):
    sims = jax.lax.dot_general(
        r.astype(jnp.bfloat16), hi, (((1,), (1,)), ((), ())),
        preferred_element_type=jnp.float32)
    m = jnp.max(sims, axis=1, keepdims=True)
    idx = jnp.min(jnp.where(sims == m, col, _K), axis=1).astype(jnp.int32)
    onehot = (col == idx[:, None]).astype(jnp.bfloat16)
    sel3 = jax.lax.dot_general(
        onehot, cbcat, (((1,), (0,)), ((), ())),
        preferred_element_type=jnp.float32)
    sel = (sel3[:, :_DIM] + sel3[:, _DIM:2 * _DIM]) + sel3[:, 2 * _DIM:]
    return r - sel, idx


def _rvq_body(q_ref, hi_ref, cbcat_ref, idx_ref, emb_ref):
    sub = _B_BLOCK // _NSPLIT
    qs = [q_ref[pl.ds(s * sub, sub), :] for s in range(_NSPLIT)]
    rs = list(qs)
    col = jax.lax.broadcasted_iota(jnp.int32, (sub, _K), 1)
    for i in range(_NCB):
        hi = hi_ref[i]
        cbcat = cbcat_ref[i]
        for s in range(_NSPLIT):
            rs[s], idx = _round(rs[s], col, hi, cbcat)
            idx_ref[i, pl.ds(s * sub, sub)] = idx
    for s in range(_NSPLIT):
        emb_ref[pl.ds(s * sub, sub), :] = qs[s] - rs[s]


@jax.jit
def kernel(query, cb0, cb1, cb2, cb3):
    B = query.shape[0]
    cbs = jnp.stack([cb0, cb1, cb2, cb3], axis=0)
    hi = cbs.astype(jnp.bfloat16)
    m16 = jnp.int32(-65536)
    t1f = jax.lax.bitcast_convert_type(
        jax.lax.bitcast_convert_type(cbs, jnp.int32) & m16, jnp.float32)
    r1 = cbs - t1f
    t2f = jax.lax.bitcast_convert_type(
        jax.lax.bitcast_convert_type(r1, jnp.int32) & m16, jnp.float32)
    r2 = r1 - t2f
    cbcat = jnp.concatenate(
        [t1f.astype(jnp.bfloat16), t2f.astype(jnp.bfloat16),
         r2.astype(jnp.bfloat16)], axis=2)
    grid = (B // _B_BLOCK,)
    idx, emb = pl.pallas_call(
        _rvq_body,
        grid=grid,
        in_specs=[
            pl.BlockSpec((_B_BLOCK, _DIM), lambda i: (i, 0)),
            pl.BlockSpec((_NCB, _K, _DIM), lambda i: (0, 0, 0)),
            pl.BlockSpec((_NCB, _K, 3 * _DIM), lambda i: (0, 0, 0)),
        ],
        out_specs=[
            pl.BlockSpec((_NCB, _B_BLOCK), lambda i: (0, i)),
            pl.BlockSpec((_B_BLOCK, _DIM), lambda i: (i, 0)),
        ],
        out_shape=[
            jax.ShapeDtypeStruct((_NCB, B), jnp.int32),
            jax.ShapeDtypeStruct((B, _DIM), jnp.float32),
        ],
        compiler_params=pltpu.CompilerParams(
            dimension_semantics=("arbitrary",),
        ),
    )(query, hi, cbcat)
    return idx, emb
